# Initial kernel scaffold; baseline (speedup 1.0000x reference)
import jax
import jax.numpy as jnp
from jax.experimental import pallas as pl


def kernel(key, keys, values):
    raise NotImplementedError("write your pallas kernel here")


# speedup vs baseline: 4.1730x; 4.1730x over previous
import functools

import jax
import jax.numpy as jnp
from jax.experimental import pallas as pl
from jax.experimental.pallas import tpu as pltpu

_DELTA = 0.001
_K = 50
_N_CHUNKS = 10
_ROW_BLK = 32
_BISECT_ITERS = 32


def _dnd_kernel(q_ref, keys_ref, vals_ref, out_ref, w_ref, *, n_chunks, k):
    j = pl.program_id(1)

    q = q_ref[...]
    kc = keys_ref[...]

    dots = jax.lax.dot_general(
        q, kc, (((1,), (1,)), ((), ())),
        preferred_element_type=jnp.float32,
        precision=jax.lax.Precision.DEFAULT,
    )
    kc2 = jax.lax.dot_general(
        jnp.ones((1, kc.shape[1]), jnp.float32), kc * kc,
        (((1,), (1,)), ((), ())),
        preferred_element_type=jnp.float32,
        precision=jax.lax.Precision.HIGHEST,
    )
    q2 = jnp.sum(q * q, axis=1, keepdims=True)
    d2 = jnp.maximum(q2 - 2.0 * dots + kc2, 0.0)
    w_ref[j] = 1.0 / (d2 + _DELTA)

    @pl.when(j == n_chunks - 1)
    def _finalize():
        rb = q.shape[0]

        def _rowmax(jj, acc):
            return jnp.maximum(acc, jnp.max(w_ref[jj], axis=1, keepdims=True))

        hi0 = jax.lax.fori_loop(
            0, n_chunks, _rowmax, jnp.zeros((rb, 1), jnp.float32))
        hi0 = hi0 * 1.001 + 1e-30
        lo0 = jnp.zeros((rb, 1), jnp.float32)

        def _count_ge(t):
            def body(jj, acc):
                m = (w_ref[jj] >= t).astype(jnp.float32)
                return acc + jnp.sum(m, axis=1, keepdims=True)
            return jax.lax.fori_loop(
                0, n_chunks, body, jnp.zeros((rb, 1), jnp.float32))

        def _bisect(_, lohi):
            lo, hi = lohi
            mid = 0.5 * (lo + hi)
            ge = _count_ge(mid) >= float(k)
            return jnp.where(ge, mid, lo), jnp.where(ge, hi, mid)

        lo, _ = jax.lax.fori_loop(0, _BISECT_ITERS, _bisect, (lo0, hi0))

        def _sums(jj, acc):
            sw, swv = acc
            wj = w_ref[jj]
            wm = jnp.where(wj >= lo, wj, 0.0)
            sw = sw + jnp.sum(wm, axis=1, keepdims=True)
            swv = swv + jnp.sum(wm * vals_ref[jj], axis=1, keepdims=True)
            return sw, swv

        zero = jnp.zeros((rb, 1), jnp.float32)
        s_w, s_wv = jax.lax.fori_loop(0, n_chunks, _sums, (zero, zero))
        out_ref[...] = s_wv / s_w


def kernel(key, keys, values):
    b, d = key.shape
    cap = keys.shape[0]
    n_chunks = _N_CHUNKS
    chunk = (-(-(cap // n_chunks) // 128)) * 128
    cap_pad = n_chunks * chunk
    keys = jnp.concatenate(
        [keys, jnp.full((cap_pad - cap, d), 1e3, jnp.float32)], axis=0)
    values = jnp.concatenate(
        [values, jnp.zeros((cap_pad - cap, 1), jnp.float32)], axis=0)
    row_blk = min(_ROW_BLK, b)

    vals_r = values.reshape(n_chunks, 1, chunk)

    out = pl.pallas_call(
        functools.partial(_dnd_kernel, n_chunks=n_chunks, k=_K),
        grid=(b // row_blk, n_chunks),
        in_specs=[
            pl.BlockSpec((row_blk, d), lambda i, j: (i, 0)),
            pl.BlockSpec((chunk, d), lambda i, j: (j, 0)),
            pl.BlockSpec((n_chunks, 1, chunk), lambda i, j: (0, 0, 0)),
        ],
        out_specs=pl.BlockSpec((row_blk, 1), lambda i, j: (i, 0)),
        out_shape=jax.ShapeDtypeStruct((b, 1), jnp.float32),
        scratch_shapes=[pltpu.VMEM((n_chunks, row_blk, chunk), jnp.float32)],
        compiler_params=pltpu.CompilerParams(
            dimension_semantics=("arbitrary", "arbitrary"),
            vmem_limit_bytes=100 * 1024 * 1024,
        ),
    )(key, keys, vals_r)
    return out
